# baseline (speedup 1.0000x reference)
import functools

import jax
import jax.numpy as jnp
from jax import lax
from jax.experimental import pallas as pl
from jax.experimental.pallas import tpu as pltpu
from jax.experimental.pallas import tpu_sc as plsc

BATCH = 16384
HIST = 200
EMBED = 32
TOTAL = BATCH * HIST
NUM_CORES = 2
NUM_SUBCORES = 16
NW = NUM_CORES * NUM_SUBCORES
BWIN = BATCH // NW
CHUNK = BWIN
NCHUNK = HIST
NBG = 3
NBS = 2
LOOK = 2
OUT_ROWS = TOTAL * EMBED // 128


def _embedding_body(table_hbm, idx_hbm, out_hbm, idx_v, rows_v, trans_v,
                    gsem, ssem):
    wid = lax.axis_index("s") * NUM_CORES + lax.axis_index("c")
    bbase = wid * BWIN
    iota16 = lax.iota(jnp.int32, 16)

    def issue_gather(h, b):
        pltpu.sync_copy(idx_hbm.at[pl.ds(h * BATCH + bbase, CHUNK)],
                        idx_v.at[b])
        pltpu.async_copy(table_hbm.at[idx_v.at[b]], rows_v.at[b], gsem.at[b])

    def wait_gather(b):
        pltpu.make_async_copy(
            table_hbm.at[idx_v.at[b]], rows_v.at[b], gsem.at[b]).wait()

    def issue_store(h, bt):
        for dblk in range(4):
            off = h * 524288 + dblk * 131072 + wid * 4096
            pltpu.async_copy(trans_v.at[bt, pl.ds(dblk * 4096, 4096)],
                             out_hbm.at[pl.ds(off, 4096)], ssem.at[bt])

    def wait_store(bt):
        for dblk in range(4):
            pltpu.make_async_copy(
                trans_v.at[bt, pl.ds(dblk * 4096, 4096)],
                out_hbm.at[pl.ds(0, 4096)], ssem.at[bt]).wait()

    d0 = iota16
    scat0 = ((d0 >> 3) * 4096 + (d0 & 7) * 128).astype(jnp.int32)
    d1 = iota16 + 16
    scat1 = ((d1 >> 3) * 4096 + (d1 & 7) * 128).astype(jnp.int32)

    def transpose(b, bt):
        trans = trans_v.at[bt]

        def step(r, carry):
            bblk = r >> 7
            br = r & 127
            base = bblk * 1024 + br
            v0 = rows_v[b, r, pl.ds(0, 16)]
            plsc.store_scatter(trans, [scat0 + base], v0)
            v1 = rows_v[b, r, pl.ds(16, 16)]
            plsc.store_scatter(trans, [scat1 + base], v1)
            return carry

        lax.fori_loop(0, CHUNK, step, 0)

    for n in range(LOOK):
        issue_gather(n, n)

    for c in range(2):
        issue_gather(c + LOOK, (c + LOOK) % NBG)
        wait_gather(c % NBG)
        transpose(c % NBG, c % NBS)
        issue_store(c, c % NBS)

    def group(g, carry):
        for k in range(6):
            c = 2 + g * 6 + k
            b = (2 + k) % NBG
            bt = k % NBS
            wait_store(bt)
            issue_gather(c + LOOK, (2 + k + LOOK) % NBG)
            wait_gather(b)
            transpose(b, bt)
            issue_store(c, bt)
        return carry

    lax.fori_loop(0, (NCHUNK - 2 - 6) // 6, group, 0)

    for c in range(NCHUNK - 6, NCHUNK):
        b = c % NBG
        bt = c % NBS
        wait_store(bt)
        if c + LOOK < NCHUNK:
            issue_gather(c + LOOK, (c + LOOK) % NBG)
        wait_gather(b)
        transpose(b, bt)
        issue_store(c, bt)

    for bt in range(NBS):
        wait_store(bt)


def kernel(indices, W):
    flat = indices.T.reshape(TOTAL).astype(jnp.int32)
    mesh = plsc.VectorSubcoreMesh(core_axis_name="c", subcore_axis_name="s")
    run = functools.partial(
        pl.kernel,
        mesh=mesh,
        out_type=jax.ShapeDtypeStruct((TOTAL * EMBED,), jnp.float32),
        scratch_types=[
            pltpu.VMEM((NBG, CHUNK), jnp.int32),
            pltpu.VMEM((NBG, CHUNK, EMBED), jnp.float32),
            pltpu.VMEM((NBS, 4 * 4096), jnp.float32),
            pltpu.SemaphoreType.DMA((NBG,)),
            pltpu.SemaphoreType.DMA((NBS,)),
        ],
        compiler_params=pltpu.CompilerParams(use_tc_tiling_on_sc=False,
                                             needs_layout_passes=False),
    )(_embedding_body)
    out = run(W, flat)
    out = out.reshape(HIST, 4, BATCH // 128, 8, 128)
    out = out.transpose(2, 4, 0, 1, 3)
    return out.reshape(BATCH, HIST, EMBED)


# speedup vs baseline: 1.2757x; 1.2757x over previous
import functools

import jax
import jax.numpy as jnp
from jax import lax
from jax.experimental import pallas as pl
from jax.experimental.pallas import tpu as pltpu
from jax.experimental.pallas import tpu_sc as plsc

BATCH = 16384
HIST = 200
EMBED = 32
TOTAL = BATCH * HIST
NUM_CORES = 2
NUM_SUBCORES = 16
NW = NUM_CORES * NUM_SUBCORES
BWIN = BATCH // NW
CHUNK = BWIN
NCHUNK = HIST
NBG = 3
NBS = 2
LOOK = 2
OUT_ROWS = TOTAL * EMBED // 128


def _embedding_body(table_hbm, idx_hbm, out_hbm, idx_v, rows_v, trans_v,
                    gsem, ssem):
    wid = lax.axis_index("s") * NUM_CORES + lax.axis_index("c")
    bbase = wid * BWIN
    iota16 = lax.iota(jnp.int32, 16)

    def issue_gather(h, b):
        pltpu.sync_copy(idx_hbm.at[pl.ds(h * BATCH + bbase, CHUNK)],
                        idx_v.at[b])
        pltpu.async_copy(table_hbm.at[idx_v.at[b]], rows_v.at[b], gsem.at[b])

    def wait_gather(b):
        pltpu.make_async_copy(
            table_hbm.at[idx_v.at[b]], rows_v.at[b], gsem.at[b]).wait()

    def issue_store(h, bt):
        for dblk in range(4):
            off = h * 524288 + dblk * 131072 + wid * 4096
            pltpu.async_copy(trans_v.at[bt, pl.ds(dblk * 4096, 4096)],
                             out_hbm.at[pl.ds(off, 4096)], ssem.at[bt])

    def wait_store(bt):
        for dblk in range(4):
            pltpu.make_async_copy(
                trans_v.at[bt, pl.ds(dblk * 4096, 4096)],
                out_hbm.at[pl.ds(0, 4096)], ssem.at[bt]).wait()

    d0 = iota16
    scat0 = ((d0 >> 3) * 4096 + (d0 & 7) * 128).astype(jnp.int32)
    d1 = iota16 + 16
    scat1 = ((d1 >> 3) * 4096 + (d1 & 7) * 128).astype(jnp.int32)

    def transpose(b, bt):
        trans = trans_v.at[bt]

        @plsc.parallel_loop(0, CHUNK, step=8, unroll=4)
        def _(r0):
            bblk = r0 >> 7
            base0 = bblk * 1024 + (r0 & 127)
            for k in range(8):
                r = r0 + k
                base = base0 + k
                v0 = rows_v[b, r, pl.ds(0, 16)]
                plsc.store_scatter(trans, [scat0 + base], v0)
                v1 = rows_v[b, r, pl.ds(16, 16)]
                plsc.store_scatter(trans, [scat1 + base], v1)

    for n in range(LOOK):
        issue_gather(n, n)

    for c in range(2):
        issue_gather(c + LOOK, (c + LOOK) % NBG)
        wait_gather(c % NBG)
        transpose(c % NBG, c % NBS)
        issue_store(c, c % NBS)

    def group(g, carry):
        for k in range(6):
            c = 2 + g * 6 + k
            b = (2 + k) % NBG
            bt = k % NBS
            wait_store(bt)
            issue_gather(c + LOOK, (2 + k + LOOK) % NBG)
            wait_gather(b)
            transpose(b, bt)
            issue_store(c, bt)
        return carry

    lax.fori_loop(0, (NCHUNK - 2 - 6) // 6, group, 0)

    for c in range(NCHUNK - 6, NCHUNK):
        b = c % NBG
        bt = c % NBS
        wait_store(bt)
        if c + LOOK < NCHUNK:
            issue_gather(c + LOOK, (c + LOOK) % NBG)
        wait_gather(b)
        transpose(b, bt)
        issue_store(c, bt)

    for bt in range(NBS):
        wait_store(bt)


def kernel(indices, W):
    flat = indices.T.reshape(TOTAL).astype(jnp.int32)
    mesh = plsc.VectorSubcoreMesh(core_axis_name="c", subcore_axis_name="s")
    run = functools.partial(
        pl.kernel,
        mesh=mesh,
        out_type=jax.ShapeDtypeStruct((TOTAL * EMBED,), jnp.float32),
        scratch_types=[
            pltpu.VMEM((NBG, CHUNK), jnp.int32),
            pltpu.VMEM((NBG, CHUNK, EMBED), jnp.float32),
            pltpu.VMEM((NBS, 4 * 4096), jnp.float32),
            pltpu.SemaphoreType.DMA((NBG,)),
            pltpu.SemaphoreType.DMA((NBS,)),
        ],
        compiler_params=pltpu.CompilerParams(use_tc_tiling_on_sc=False,
                                             needs_layout_passes=False),
    )(_embedding_body)
    out = run(W, flat)
    out = out.reshape(HIST, 4, BATCH // 128, 8, 128)
    out = out.transpose(2, 4, 0, 1, 3)
    return out.reshape(BATCH, HIST, EMBED)


# speedup vs baseline: 2.2410x; 1.7567x over previous
import functools

import jax
import jax.numpy as jnp
from jax import lax
from jax.experimental import pallas as pl
from jax.experimental.pallas import tpu as pltpu
from jax.experimental.pallas import tpu_sc as plsc

BATCH = 16384
HIST = 200
EMBED = 32
TOTAL = BATCH * HIST
NUM_CORES = 2
NUM_SUBCORES = 16
NW = NUM_CORES * NUM_SUBCORES
BWIN = BATCH // NW
CHUNK = BWIN
NCHUNK = HIST
NBG = 3
NBS = 2
LOOK = 2
OUT_ROWS = TOTAL * EMBED // 128


def _embedding_body(table_hbm, idx_hbm, out_hbm, idx_v, rows_v, trans_v,
                    gsem, ssem):
    wid = lax.axis_index("s") * NUM_CORES + lax.axis_index("c")
    bbase = wid * BWIN
    iota16 = lax.iota(jnp.int32, 16)

    def issue_gather(h, b):
        pltpu.sync_copy(idx_hbm.at[pl.ds(h * BATCH + bbase, CHUNK)],
                        idx_v.at[b])
        pltpu.async_copy(table_hbm.at[idx_v.at[b]], rows_v.at[b], gsem.at[b])

    def wait_gather(b):
        pltpu.make_async_copy(
            table_hbm.at[idx_v.at[b]], rows_v.at[b], gsem.at[b]).wait()

    def issue_store(h, bt):
        for dblk in range(4):
            rowbase = h * 4096 + dblk * 1024 + wid * 32
            pltpu.async_copy(
                trans_v.at[bt, pl.ds(dblk * 40, 32), pl.ds(0, 128)],
                out_hbm.at[pl.ds(rowbase, 32), :], ssem.at[bt])

    def wait_store(bt):
        for dblk in range(4):
            pltpu.make_async_copy(
                trans_v.at[bt, pl.ds(dblk * 40, 32), pl.ds(0, 128)],
                out_hbm.at[pl.ds(0, 32), :], ssem.at[bt]).wait()

    row2c0 = ((iota16 >> 3) * 40 + (iota16 & 7)).astype(jnp.int32)
    row2c1 = row2c0 + 80

    def transpose(b, bt):
        trans = trans_v.at[bt]

        @plsc.parallel_loop(0, CHUNK, step=8, unroll=4)
        def _(r0):
            bblk = r0 >> 7
            roff = bblk * 8
            for k in range(8):
                r = r0 + k
                col = jnp.full((16,), r & 127, jnp.int32)
                v0 = rows_v[b, r, pl.ds(0, 16)]
                plsc.store_scatter(trans, [row2c0 + roff, col], v0)
                v1 = rows_v[b, r, pl.ds(16, 16)]
                plsc.store_scatter(trans, [row2c1 + roff, col], v1)

    for n in range(LOOK):
        issue_gather(n, n)

    for c in range(2):
        issue_gather(c + LOOK, (c + LOOK) % NBG)
        wait_gather(c % NBG)
        transpose(c % NBG, c % NBS)
        issue_store(c, c % NBS)

    def group(g, carry):
        for k in range(6):
            c = 2 + g * 6 + k
            b = (2 + k) % NBG
            bt = k % NBS
            wait_store(bt)
            issue_gather(c + LOOK, (2 + k + LOOK) % NBG)
            wait_gather(b)
            transpose(b, bt)
            issue_store(c, bt)
        return carry

    lax.fori_loop(0, (NCHUNK - 2 - 6) // 6, group, 0)

    for c in range(NCHUNK - 6, NCHUNK):
        b = c % NBG
        bt = c % NBS
        wait_store(bt)
        if c + LOOK < NCHUNK:
            issue_gather(c + LOOK, (c + LOOK) % NBG)
        wait_gather(b)
        transpose(b, bt)
        issue_store(c, bt)

    for bt in range(NBS):
        wait_store(bt)


def kernel(indices, W):
    flat = indices.T.reshape(TOTAL).astype(jnp.int32)
    mesh = plsc.VectorSubcoreMesh(core_axis_name="c", subcore_axis_name="s")
    run = functools.partial(
        pl.kernel,
        mesh=mesh,
        out_type=jax.ShapeDtypeStruct((OUT_ROWS, 128), jnp.float32),
        scratch_types=[
            pltpu.VMEM((NBG, CHUNK), jnp.int32),
            pltpu.VMEM((NBG, CHUNK, EMBED), jnp.float32),
            pltpu.VMEM((NBS, 160, 129), jnp.float32),
            pltpu.SemaphoreType.DMA((NBG,)),
            pltpu.SemaphoreType.DMA((NBS,)),
        ],
        compiler_params=pltpu.CompilerParams(use_tc_tiling_on_sc=False,
                                             needs_layout_passes=False),
    )(_embedding_body)
    out = run(W, flat)
    out = out.reshape(HIST, 4, BATCH // 128, 8, 128)
    out = out.transpose(2, 4, 0, 1, 3)
    return out.reshape(BATCH, HIST, EMBED)
